# baseline (speedup 1.0000x reference)
import functools

import jax
import jax.numpy as jnp
import numpy as np
from jax.experimental import pallas as pl
from jax.experimental.pallas import tpu as pltpu

_S = 5
_SUBW = 512
_TINY = np.float32(np.finfo(np.float32).tiny)
_KEY_HI = 0
_KEY_LO = 42


def _rotl(v, d):
    return (v << jnp.uint32(d)) | (v >> jnp.uint32(32 - d))


def _threefry_bits_multi(i_list):
    ks1 = jnp.uint32(_KEY_LO)
    ks2 = jnp.uint32(_KEY_HI ^ _KEY_LO ^ 0x1BD11BDA)
    rot1 = (13, 15, 26, 6)
    rot2 = (17, 29, 16, 24)

    def step(ps, d):
        adds = [x0 + x1 for (x0, x1) in ps]
        shls = [x1 << jnp.uint32(d) for (_, x1) in ps]
        shrs = [x1 >> jnp.uint32(32 - d) for (_, x1) in ps]
        ors = [a | b for a, b in zip(shls, shrs)]
        xors = [o ^ a for o, a in zip(ors, adds)]
        return list(zip(adds, xors))

    def group(ps, rots):
        for d in rots:
            ps = step(ps, d)
        return ps

    ps = []
    for i_u32 in i_list:
        x1 = i_u32 + ks1
        x0 = x1
        x1 = _rotl(x1, rot1[0]) ^ x0
        ps.append((x0, x1))
    ps = group(ps, rot1[1:])
    ps = [(x0 + ks1, x1 + (ks2 + jnp.uint32(1))) for x0, x1 in ps]
    ps = group(ps, rot2)
    ps = [(x0 + ks2, x1 + jnp.uint32(2)) for x0, x1 in ps]
    ps = group(ps, rot1)
    ps = [(x0, x1 + (ks1 + jnp.uint32(3))) for x0, x1 in ps]
    ps = group(ps, rot2)
    ps = [(x0 + ks1, x1 + (ks2 + jnp.uint32(4))) for x0, x1 in ps]
    ps = group(ps, rot1)
    ps = [(x0 + ks2, x1 + jnp.uint32(5)) for x0, x1 in ps]
    return [x0 ^ x1 for x0, x1 in ps]


def _threefry_from_x1(x1):
    ks1 = jnp.uint32(_KEY_LO)
    ks2 = jnp.uint32(_KEY_HI ^ _KEY_LO ^ 0x1BD11BDA)
    rot1 = (13, 15, 26, 6)
    rot2 = (17, 29, 16, 24)

    def rounds(x0, x1, rots):
        for d in rots:
            x0 = x0 + x1
            x1 = _rotl(x1, d)
            x1 = x1 ^ x0
        return x0, x1

    x0 = x1
    x1 = _rotl(x1, rot1[0]) ^ x0
    x0, x1 = rounds(x0, x1, rot1[1:])
    x0 = x0 + ks1
    x1 = x1 + (ks2 + jnp.uint32(1))
    x0, x1 = rounds(x0, x1, rot2)
    x0 = x0 + ks2
    x1 = x1 + jnp.uint32(2)
    x0, x1 = rounds(x0, x1, rot1)
    x1 = x1 + (ks1 + jnp.uint32(3))
    x0, x1 = rounds(x0, x1, rot2)
    x0 = x0 + ks1
    x1 = x1 + (ks2 + jnp.uint32(4))
    x0, x1 = rounds(x0, x1, rot1)
    x0 = x0 + ks2
    x1 = x1 + jnp.uint32(5)
    return x0 ^ x1


def _threefry_bits(i_u32):
    ks0 = jnp.uint32(_KEY_HI)
    ks1 = jnp.uint32(_KEY_LO)
    ks2 = jnp.uint32(_KEY_HI ^ _KEY_LO ^ 0x1BD11BDA)
    rot1 = (13, 15, 26, 6)
    rot2 = (17, 29, 16, 24)

    def rounds(x0, x1, rots):
        for d in rots:
            x0 = x0 + x1
            x1 = _rotl(x1, d)
            x1 = x1 ^ x0
        return x0, x1

    x1 = i_u32 + ks1
    x0 = x1
    x1 = _rotl(x1, rot1[0]) ^ x0
    x0, x1 = rounds(x0, x1, rot1[1:])
    x0 = x0 + ks1
    x1 = x1 + (ks2 + jnp.uint32(1))
    x0, x1 = rounds(x0, x1, rot2)
    x0 = x0 + ks2
    x1 = x1 + jnp.uint32(2)
    x0, x1 = rounds(x0, x1, rot1)
    x1 = x1 + (ks1 + jnp.uint32(3))
    x0, x1 = rounds(x0, x1, rot2)
    x0 = x0 + ks1
    x1 = x1 + (ks2 + jnp.uint32(4))
    x0, x1 = rounds(x0, x1, rot1)
    x0 = x0 + ks2
    x1 = x1 + jnp.uint32(5)
    return x0 ^ x1


def _neg_gumbel_from_bits(bits):
    fb = (bits >> jnp.uint32(9)) | jnp.uint32(0x3F800000)
    floats = jax.lax.bitcast_convert_type(fb, jnp.float32) - jnp.float32(1.0)
    u = floats + jnp.float32(_TINY)
    t = jnp.log2(u) * jnp.float32(-0.6931471805599453)
    return jnp.log(t)


def _body(x_ref, out_ref, vmax_ref, vidx_ref, *,
          vocab, n_sub, chunk, n_chunks):
    r = pl.program_id(0)
    c = pl.program_id(1)
    per_sub = vocab // n_sub
    first = c == 0

    sub = jax.lax.broadcasted_iota(jnp.uint32, (n_sub, chunk), 0)
    col = jax.lax.broadcasted_iota(jnp.uint32, (n_sub, chunk), 1)
    cglob = jnp.uint32(c * chunk) + col
    jglob = sub * jnp.uint32(per_sub) + cglob
    logits = jnp.log(x_ref[0] + jnp.float32(1e-30))
    lm = jnp.where(cglob < jnp.uint32(per_sub), logits, jnp.float32(-3.0e38))

    @pl.when(first)
    def _():
        for s in range(_S):
            vmax_ref[s] = jnp.full((n_sub, chunk), -3.1e38, jnp.float32)

    for s in range(_S):
        base = (r * _S + s) * vocab
        x1 = jglob + jnp.uint32(base + _KEY_LO)
        ng = _neg_gumbel_from_bits(_threefry_from_x1(x1))
        val = lm - ng

        better = val > vmax_ref[s]
        vmax_ref[s] = jnp.where(better, val, vmax_ref[s])
        vidx_ref[s] = jnp.where(better, jglob.astype(jnp.int32),
                                vidx_ref[s])

    @pl.when(c == n_chunks - 1)
    def _():
        for s in range(_S):
            vm = vmax_ref[s]
            m = jnp.max(vm)
            idx = jnp.min(jnp.where(vm == m, vidx_ref[s],
                                    jnp.int32(0x7FFFFFFF)))
            out_ref[0, 0, s] = idx


@jax.jit
def kernel(x):
    b, vocab = x.shape
    n_sub = 8
    per_sub = vocab // n_sub
    assert per_sub * n_sub == vocab
    chunk = 6144
    n_chunks = -(-per_sub // chunk)

    x3 = x.reshape(b, n_sub, per_sub)
    out = pl.pallas_call(
        functools.partial(_body, vocab=vocab, n_sub=n_sub, chunk=chunk,
                          n_chunks=n_chunks),
        grid=(b, n_chunks),
        in_specs=[
            pl.BlockSpec((1, n_sub, chunk), lambda r, c: (r, 0, c)),
        ],
        out_specs=pl.BlockSpec((1, 1, _S), lambda r, c: (r, 0, 0),
                               memory_space=pltpu.SMEM),
        out_shape=jax.ShapeDtypeStruct((b, 1, _S), jnp.int32),
        scratch_shapes=[
            pltpu.VMEM((_S, n_sub, chunk), jnp.float32),
            pltpu.VMEM((_S, n_sub, chunk), jnp.int32),
        ],
    )(x3)
    return out.reshape(b, _S).astype(jnp.int64)


# speedup vs baseline: 1.0063x; 1.0063x over previous
import functools

import jax
import jax.numpy as jnp
import numpy as np
from jax.experimental import pallas as pl
from jax.experimental.pallas import tpu as pltpu

_S = 5
_SUBW = 512
_TINY = np.float32(np.finfo(np.float32).tiny)
_KEY_HI = 0
_KEY_LO = 42


def _rotl(v, d):
    return (v << jnp.uint32(d)) | (v >> jnp.uint32(32 - d))


def _threefry_bits_multi(i_list):
    ks1 = jnp.uint32(_KEY_LO)
    ks2 = jnp.uint32(_KEY_HI ^ _KEY_LO ^ 0x1BD11BDA)
    rot1 = (13, 15, 26, 6)
    rot2 = (17, 29, 16, 24)

    def step(ps, d):
        adds = [x0 + x1 for (x0, x1) in ps]
        shls = [x1 << jnp.uint32(d) for (_, x1) in ps]
        shrs = [x1 >> jnp.uint32(32 - d) for (_, x1) in ps]
        ors = [a | b for a, b in zip(shls, shrs)]
        xors = [o ^ a for o, a in zip(ors, adds)]
        return list(zip(adds, xors))

    def group(ps, rots):
        for d in rots:
            ps = step(ps, d)
        return ps

    ps = []
    for i_u32 in i_list:
        x1 = i_u32 + ks1
        x0 = x1
        x1 = _rotl(x1, rot1[0]) ^ x0
        ps.append((x0, x1))
    ps = group(ps, rot1[1:])
    ps = [(x0 + ks1, x1 + (ks2 + jnp.uint32(1))) for x0, x1 in ps]
    ps = group(ps, rot2)
    ps = [(x0 + ks2, x1 + jnp.uint32(2)) for x0, x1 in ps]
    ps = group(ps, rot1)
    ps = [(x0, x1 + (ks1 + jnp.uint32(3))) for x0, x1 in ps]
    ps = group(ps, rot2)
    ps = [(x0 + ks1, x1 + (ks2 + jnp.uint32(4))) for x0, x1 in ps]
    ps = group(ps, rot1)
    ps = [(x0 + ks2, x1 + jnp.uint32(5)) for x0, x1 in ps]
    return [x0 ^ x1 for x0, x1 in ps]


def _threefry_from_x1(x1):
    ks1 = jnp.uint32(_KEY_LO)
    ks2 = jnp.uint32(_KEY_HI ^ _KEY_LO ^ 0x1BD11BDA)
    rot1 = (13, 15, 26, 6)
    rot2 = (17, 29, 16, 24)

    def rounds(x0, x1, rots):
        for d in rots:
            x0 = x0 + x1
            x1 = _rotl(x1, d)
            x1 = x1 ^ x0
        return x0, x1

    x0 = x1
    x1 = _rotl(x1, rot1[0]) ^ x0
    x0, x1 = rounds(x0, x1, rot1[1:])
    x0 = x0 + ks1
    x1 = x1 + (ks2 + jnp.uint32(1))
    x0, x1 = rounds(x0, x1, rot2)
    x0 = x0 + ks2
    x1 = x1 + jnp.uint32(2)
    x0, x1 = rounds(x0, x1, rot1)
    x1 = x1 + (ks1 + jnp.uint32(3))
    x0, x1 = rounds(x0, x1, rot2)
    x0 = x0 + ks1
    x1 = x1 + (ks2 + jnp.uint32(4))
    x0, x1 = rounds(x0, x1, rot1)
    x0 = x0 + ks2
    x1 = x1 + jnp.uint32(5)
    return x0 ^ x1


def _threefry_bits(i_u32):
    ks0 = jnp.uint32(_KEY_HI)
    ks1 = jnp.uint32(_KEY_LO)
    ks2 = jnp.uint32(_KEY_HI ^ _KEY_LO ^ 0x1BD11BDA)
    rot1 = (13, 15, 26, 6)
    rot2 = (17, 29, 16, 24)

    def rounds(x0, x1, rots):
        for d in rots:
            x0 = x0 + x1
            x1 = _rotl(x1, d)
            x1 = x1 ^ x0
        return x0, x1

    x1 = i_u32 + ks1
    x0 = x1
    x1 = _rotl(x1, rot1[0]) ^ x0
    x0, x1 = rounds(x0, x1, rot1[1:])
    x0 = x0 + ks1
    x1 = x1 + (ks2 + jnp.uint32(1))
    x0, x1 = rounds(x0, x1, rot2)
    x0 = x0 + ks2
    x1 = x1 + jnp.uint32(2)
    x0, x1 = rounds(x0, x1, rot1)
    x1 = x1 + (ks1 + jnp.uint32(3))
    x0, x1 = rounds(x0, x1, rot2)
    x0 = x0 + ks1
    x1 = x1 + (ks2 + jnp.uint32(4))
    x0, x1 = rounds(x0, x1, rot1)
    x0 = x0 + ks2
    x1 = x1 + jnp.uint32(5)
    return x0 ^ x1


def _neg_gumbel_from_bits(bits):
    fb = (bits >> jnp.uint32(9)) | jnp.uint32(0x3F800000)
    floats = jax.lax.bitcast_convert_type(fb, jnp.float32) - jnp.float32(1.0)
    u = floats + jnp.float32(_TINY)
    return jnp.log(-jnp.log(u))


def _body(x_ref, out_ref, vmax_ref, vidx_ref, *,
          vocab, n_sub, chunk, n_chunks):
    r = pl.program_id(0)
    c = pl.program_id(1)
    per_sub = vocab // n_sub
    first = c == 0

    sub = jax.lax.broadcasted_iota(jnp.uint32, (n_sub, chunk), 0)
    col = jax.lax.broadcasted_iota(jnp.uint32, (n_sub, chunk), 1)
    cglob = jnp.uint32(c * chunk) + col
    jglob = sub * jnp.uint32(per_sub) + cglob
    logits = jnp.log(x_ref[0] + jnp.float32(1e-30))
    lm = jnp.where(cglob < jnp.uint32(per_sub), logits, jnp.float32(-3.0e38))

    @pl.when(first)
    def _():
        for s in range(_S):
            vmax_ref[s] = jnp.full((n_sub, chunk), -3.1e38, jnp.float32)

    for s in range(_S):
        base = (r * _S + s) * vocab
        x1 = jglob + jnp.uint32(base + _KEY_LO)
        ng = _neg_gumbel_from_bits(_threefry_from_x1(x1))
        val = lm - ng

        better = val > vmax_ref[s]
        vmax_ref[s] = jnp.where(better, val, vmax_ref[s])
        vidx_ref[s] = jnp.where(better, jglob.astype(jnp.int32),
                                vidx_ref[s])

    @pl.when(c == n_chunks - 1)
    def _():
        for s in range(_S):
            vm = vmax_ref[s]
            m = jnp.max(vm)
            idx = jnp.min(jnp.where(vm == m, vidx_ref[s],
                                    jnp.int32(0x7FFFFFFF)))
            out_ref[0, 0, s] = idx


@jax.jit
def kernel(x):
    b, vocab = x.shape
    n_sub = 8
    per_sub = vocab // n_sub
    assert per_sub * n_sub == vocab
    chunk = 6144
    n_chunks = -(-per_sub // chunk)

    x3 = x.reshape(b, n_sub, per_sub)
    out = pl.pallas_call(
        functools.partial(_body, vocab=vocab, n_sub=n_sub, chunk=chunk,
                          n_chunks=n_chunks),
        grid=(b, n_chunks),
        in_specs=[
            pl.BlockSpec((1, n_sub, chunk), lambda r, c: (r, 0, c)),
        ],
        out_specs=pl.BlockSpec((1, 1, _S), lambda r, c: (r, 0, 0),
                               memory_space=pltpu.SMEM),
        out_shape=jax.ShapeDtypeStruct((b, 1, _S), jnp.int32),
        scratch_shapes=[
            pltpu.VMEM((_S, n_sub, chunk), jnp.float32),
            pltpu.VMEM((_S, n_sub, chunk), jnp.int32),
        ],
    )(x3)
    return out.reshape(b, _S).astype(jnp.int64)
